# baseline (speedup 1.0000x reference)
import functools

import jax
import jax.numpy as jnp
from jax import lax
from jax.experimental import pallas as pl
from jax.experimental.pallas import tpu as pltpu
from jax.experimental.pallas import tpu_sc as plsc

M = 100000
D = 128
N = 16384
B = 16384

NC = 2
NS = 16
L = 16
NW = NC * NS

SPT = 3136
STAMP_N = SPT * NW + 128
CH = 128
CHUNK = N // NS
KCH = CHUNK // CH
SLOTS = N // NC
DUMP = SLOTS
ACC_PT = 520
ACC_ROWS = ACC_PT * NS
ZROWS = ACC_PT // 4

_mesh = plsc.VectorSubcoreMesh(core_axis_name="c", subcore_axis_name="s")


def _stamp_body(dst_hbm, zero_hbm, stamp_hbm, dstv, zv, tgt2d, val1d):
    c = lax.axis_index("c")
    s = lax.axis_index("s")
    wid = s * NC + c
    base = wid * SPT
    pltpu.sync_copy(zero_hbm, zv)
    pltpu.sync_copy(zv, stamp_hbm.at[pl.ds(base, SPT)])
    pltpu.sync_copy(dst_hbm, dstv)
    ii = lax.iota(jnp.int32, L)

    def comp(v, carry):
        ids = dstv[pl.ds(v * L, L)]
        d = ids - base
        outr = (d | (SPT - 1 - d)) >> 31
        r, col = v // 8, (v % 8) * L
        tgt2d[r, pl.ds(col, L)] = ids * (1 + outr) - (SPT * NW) * outr
        val1d[pl.ds(v * L, L)] = v * L + ii + 1
        return carry

    lax.fori_loop(0, N // L, comp, jnp.int32(0), unroll=8)

    def scat(k, carry):
        pltpu.sync_copy(val1d.at[pl.ds(k * CH, CH)], stamp_hbm.at[tgt2d.at[k]])
        return carry

    lax.fori_loop(0, N // CH, scat, jnp.int32(0))


_stamp_call = functools.partial(
    pl.kernel,
    out_type=jax.ShapeDtypeStruct((STAMP_N,), jnp.int32),
    mesh=_mesh,
    scratch_types=[
        pltpu.VMEM((N,), jnp.int32),
        pltpu.VMEM((SPT,), jnp.int32),
        pltpu.VMEM((N // CH, CH), jnp.int32),
        pltpu.VMEM((N,), jnp.int32),
    ],
)(_stamp_body)


def _main_body(mem_hbm, msgs_hbm, dst_hbm, q_hbm, stamp_hbm, zf_hbm, out_hbm,
               dstv, qv, slotm, sq, tgtm, gsrc, oposa, oposb, rowa, rowb, zbuf, acc):
    c = lax.axis_index("c")
    s = lax.axis_index("s")
    pltpu.sync_copy(zf_hbm, zbuf)
    r0 = s * ACC_PT
    for i in range(4):
        pltpu.sync_copy(zbuf, acc.at[pl.ds(r0 + i * ZROWS, ZROWS)])

    cb = s * CHUNK
    pltpu.sync_copy(dst_hbm.at[pl.ds(cb, CHUNK)], dstv)
    pltpu.sync_copy(q_hbm.at[pl.ds(cb, CHUNK)], qv)
    for k in range(KCH):
        pltpu.sync_copy(stamp_hbm.at[dstv.at[pl.ds(k * CH, CH)]],
                        slotm.at[pl.ds(k * CH, CH)])
        pltpu.sync_copy(stamp_hbm.at[qv.at[pl.ds(k * CH, CH)]],
                        sq.at[pl.ds(k * CH, CH)])

    ii = lax.iota(jnp.int32, L)
    dumpv = jnp.full((L,), DUMP, jnp.int32)
    dumpo = jnp.full((L,), B, jnp.int32)

    def tcomp(v, carry):
        r, col = v // 8, (v % 8) * L
        st = slotm[pl.ds(v * L, L)] - 1
        mm = 1 - ((st ^ c) & 1)
        tgtm[r, pl.ds(col, L)] = (st >> 1) * mm + DUMP * (1 - mm)
        sv = sq[pl.ds(v * L, L)]
        q = qv[pl.ds(v * L, L)]
        stq = sv - 1
        pres = jnp.minimum(sv, 1)
        pos = cb + v * L + ii
        pm = pres * (1 - ((stq ^ c) & 1))
        gsrc[r, pl.ds(col, L)] = (stq >> 1) * pm + DUMP * (1 - pm)
        oposa[r, pl.ds(col, L)] = pos * pm + B * (1 - pm)
        am = (1 - pres) * (1 - ((q ^ c) & 1))
        oposb[r, pl.ds(col, L)] = pos * am + B * (1 - am)
        return carry

    lax.fori_loop(0, CHUNK // L, tcomp, jnp.int32(0), unroll=8)
    plsc.subcore_barrier()

    def madd(k, carry):
        pltpu.sync_copy(msgs_hbm.at[pl.ds(cb + k * CH, CH)], rowa)
        pltpu.sync_copy(rowa, acc.at[tgtm.at[k]], add=True)
        return carry

    lax.fori_loop(0, KCH, madd, jnp.int32(0))
    plsc.subcore_barrier()

    def pout(k, carry):
        pltpu.sync_copy(acc.at[gsrc.at[k]], rowa)
        pltpu.sync_copy(rowa, out_hbm.at[oposa.at[k]])
        return carry

    lax.fori_loop(0, KCH, pout, jnp.int32(0))

    def aout(k, carry):
        pltpu.sync_copy(mem_hbm.at[qv.at[pl.ds(k * CH, CH)]], rowb)
        pltpu.sync_copy(rowb, out_hbm.at[oposb.at[k]])
        return carry

    lax.fori_loop(0, KCH, aout, jnp.int32(0))


_main_call = functools.partial(
    pl.kernel,
    out_type=jax.ShapeDtypeStruct((B + CH, D), jnp.float32),
    mesh=_mesh,
    scratch_types=[
        pltpu.VMEM((CHUNK,), jnp.int32),
        pltpu.VMEM((CHUNK,), jnp.int32),
        pltpu.VMEM((CHUNK,), jnp.int32),
        pltpu.VMEM((CHUNK,), jnp.int32),
        pltpu.VMEM((KCH, CH), jnp.int32),
        pltpu.VMEM((KCH, CH), jnp.int32),
        pltpu.VMEM((KCH, CH), jnp.int32),
        pltpu.VMEM((KCH, CH), jnp.int32),
        pltpu.VMEM((CH, D), jnp.float32),
        pltpu.VMEM((CH, D), jnp.float32),
        pltpu.VMEM((ZROWS, D), jnp.float32),
        pltpu.VMEM_SHARED((ACC_ROWS, D), jnp.float32),
    ],
)(_main_body)


def kernel(mem, msgs, dst_ids, query_ids):
    dst_ids = dst_ids.astype(jnp.int32)
    query_ids = query_ids.astype(jnp.int32)
    zi = jnp.zeros((SPT,), jnp.int32)
    zf = jnp.zeros((ZROWS, D), jnp.float32)
    stamp = _stamp_call(dst_ids, zi)
    out = _main_call(mem, msgs, dst_ids, query_ids, stamp, zf)
    return out[:B]


# speedup vs baseline: 465.9566x; 465.9566x over previous
import functools

import jax
import jax.numpy as jnp
from jax import lax
from jax.experimental import pallas as pl
from jax.experimental.pallas import tpu as pltpu
from jax.experimental.pallas import tpu_sc as plsc

M = 100000
D = 128
N = 16384
B = 16384

NC = 2
NS = 16
L = 16
NW = NC * NS

SPT = 3136
STAMP_N = SPT * NW
CH = 128
CHUNK = N // NS
KCH = CHUNK // CH
SLOTS = N // NC
DUMP = SLOTS
ACC_PT = 520
ACC_ROWS = ACC_PT * NS
ZROWS = ACC_PT // 4

_mesh = plsc.VectorSubcoreMesh(core_axis_name="c", subcore_axis_name="s")


def _stamp_body(dst_hbm, stamp_hbm, dstv, stampv):
    c = lax.axis_index("c")
    s = lax.axis_index("s")
    wid = s * NC + c
    base = wid * SPT
    zero = jnp.zeros((L,), jnp.int32)

    def z(i, carry):
        stampv[pl.ds(i * L, L)] = zero
        return carry

    lax.fori_loop(0, SPT // L, z, jnp.int32(0), unroll=8)
    pltpu.sync_copy(dst_hbm, dstv)
    ii = lax.iota(jnp.int32, L)

    def comp(v, carry):
        ids = dstv[pl.ds(v * L, L)]
        d = ids - base
        m = (d >= 0) & (d < SPT)
        dcl = jnp.minimum(jnp.maximum(d, 0), SPT - 1)
        plsc.store_scatter(stampv, [dcl], v * L + ii + 1, mask=m)
        return carry

    lax.fori_loop(0, N // L, comp, jnp.int32(0), unroll=8)
    pltpu.sync_copy(stampv, stamp_hbm.at[pl.ds(base, SPT)])


_stamp_call = functools.partial(
    pl.kernel,
    out_type=jax.ShapeDtypeStruct((STAMP_N,), jnp.int32),
    mesh=_mesh,
    compiler_params=pltpu.CompilerParams(needs_layout_passes=False),
    scratch_types=[
        pltpu.VMEM((N,), jnp.int32),
        pltpu.VMEM((SPT,), jnp.int32),
    ],
)(_stamp_body)


def _main_body(mem_hbm, msgs_hbm, dst_hbm, q_hbm, stamp_hbm, zf_hbm, out_hbm,
               dstv, qv, slotm, sq, tgtm, gsrc, oposa, oposb, rowa, rowb, zbuf, acc):
    c = lax.axis_index("c")
    s = lax.axis_index("s")
    pltpu.sync_copy(zf_hbm, zbuf)
    r0 = s * ACC_PT
    for i in range(4):
        pltpu.sync_copy(zbuf, acc.at[pl.ds(r0 + i * ZROWS, ZROWS)])

    cb = s * CHUNK
    pltpu.sync_copy(dst_hbm.at[pl.ds(cb, CHUNK)], dstv)
    pltpu.sync_copy(q_hbm.at[pl.ds(cb, CHUNK)], qv)
    for k in range(KCH):
        pltpu.sync_copy(stamp_hbm.at[dstv.at[pl.ds(k * CH, CH)]],
                        slotm.at[pl.ds(k * CH, CH)])
        pltpu.sync_copy(stamp_hbm.at[qv.at[pl.ds(k * CH, CH)]],
                        sq.at[pl.ds(k * CH, CH)])

    ii = lax.iota(jnp.int32, L)
    wid = s * NC + c
    dmp = DUMP + s
    odmp = B + wid

    def tcomp(v, carry):
        r, col = v // 8, (v % 8) * L
        st = slotm[pl.ds(v * L, L)] - 1
        mm = 1 - ((st ^ c) & 1)
        tgtm[r, pl.ds(col, L)] = (st >> 1) * mm + dmp * (1 - mm)
        sv = sq[pl.ds(v * L, L)]
        q = qv[pl.ds(v * L, L)]
        stq = sv - 1
        pres = jnp.minimum(sv, 1)
        pos = cb + v * L + ii
        pm = pres * (1 - ((stq ^ c) & 1))
        gsrc[r, pl.ds(col, L)] = (stq >> 1) * pm + dmp * (1 - pm)
        oposa[r, pl.ds(col, L)] = pos * pm + odmp * (1 - pm)
        am = (1 - pres) * (1 - ((q ^ c) & 1))
        oposb[r, pl.ds(col, L)] = pos * am + odmp * (1 - am)
        return carry

    lax.fori_loop(0, CHUNK // L, tcomp, jnp.int32(0), unroll=8)
    plsc.subcore_barrier()

    def madd(k, carry):
        pltpu.sync_copy(msgs_hbm.at[pl.ds(cb + k * CH, CH)], rowa)
        pltpu.sync_copy(rowa, acc.at[tgtm.at[k]], add=True)
        return carry

    lax.fori_loop(0, KCH, madd, jnp.int32(0))
    plsc.subcore_barrier()

    def pout(k, carry):
        pltpu.sync_copy(acc.at[gsrc.at[k]], rowa)
        pltpu.sync_copy(rowa, out_hbm.at[oposa.at[k]])
        return carry

    lax.fori_loop(0, KCH, pout, jnp.int32(0))

    def aout(k, carry):
        pltpu.sync_copy(mem_hbm.at[qv.at[pl.ds(k * CH, CH)]], rowb)
        pltpu.sync_copy(rowb, out_hbm.at[oposb.at[k]])
        return carry

    lax.fori_loop(0, KCH, aout, jnp.int32(0))


_main_call = functools.partial(
    pl.kernel,
    out_type=jax.ShapeDtypeStruct((B + CH, D), jnp.float32),
    mesh=_mesh,
    scratch_types=[
        pltpu.VMEM((CHUNK,), jnp.int32),
        pltpu.VMEM((CHUNK,), jnp.int32),
        pltpu.VMEM((CHUNK,), jnp.int32),
        pltpu.VMEM((CHUNK,), jnp.int32),
        pltpu.VMEM((KCH, CH), jnp.int32),
        pltpu.VMEM((KCH, CH), jnp.int32),
        pltpu.VMEM((KCH, CH), jnp.int32),
        pltpu.VMEM((KCH, CH), jnp.int32),
        pltpu.VMEM((CH, D), jnp.float32),
        pltpu.VMEM((CH, D), jnp.float32),
        pltpu.VMEM((ZROWS, D), jnp.float32),
        pltpu.VMEM_SHARED((ACC_ROWS, D), jnp.float32),
    ],
)(_main_body)


def kernel(mem, msgs, dst_ids, query_ids):
    dst_ids = dst_ids.astype(jnp.int32)
    query_ids = query_ids.astype(jnp.int32)
    zf = jnp.zeros((ZROWS, D), jnp.float32)
    stamp = _stamp_call(dst_ids)
    out = _main_call(mem, msgs, dst_ids, query_ids, stamp, zf)
    return out[:B]


# speedup vs baseline: 523.7545x; 1.1240x over previous
import functools

import jax
import jax.numpy as jnp
from jax import lax
from jax.experimental import pallas as pl
from jax.experimental.pallas import tpu as pltpu
from jax.experimental.pallas import tpu_sc as plsc

M = 100000
D = 128
N = 16384
B = 16384

NC = 2
NS = 16
L = 16
NW = NC * NS

SPT = 3136
STAMP_N = SPT * NW
CH = 64
CHUNK = N // NS
KCH = CHUNK // CH
SLOTS = N // NC
DUMP = SLOTS
ACC_PT = 520
ACC_ROWS = ACC_PT * NS
ZROWS = ACC_PT // 4

_mesh = plsc.VectorSubcoreMesh(core_axis_name="c", subcore_axis_name="s")


def _stamp_body(dst_hbm, stamp_hbm, dstv, stampv):
    c = lax.axis_index("c")
    s = lax.axis_index("s")
    wid = s * NC + c
    base = wid * SPT
    zero = jnp.zeros((L,), jnp.int32)

    def z(i, carry):
        stampv[pl.ds(i * L, L)] = zero
        return carry

    lax.fori_loop(0, SPT // L, z, jnp.int32(0), unroll=8)
    pltpu.sync_copy(dst_hbm, dstv)
    ii = lax.iota(jnp.int32, L)

    def comp(v, carry):
        ids = dstv[pl.ds(v * L, L)]
        d = ids - base
        m = (d >= 0) & (d < SPT)
        dcl = jnp.minimum(jnp.maximum(d, 0), SPT - 1)
        plsc.store_scatter(stampv, [dcl], v * L + ii + 1, mask=m)
        return carry

    lax.fori_loop(0, N // L, comp, jnp.int32(0), unroll=8)
    pltpu.sync_copy(stampv, stamp_hbm.at[pl.ds(base, SPT)])


_stamp_call = functools.partial(
    pl.kernel,
    out_type=jax.ShapeDtypeStruct((STAMP_N,), jnp.int32),
    mesh=_mesh,
    compiler_params=pltpu.CompilerParams(needs_layout_passes=False),
    scratch_types=[
        pltpu.VMEM((N,), jnp.int32),
        pltpu.VMEM((SPT,), jnp.int32),
    ],
)(_stamp_body)


def _main_body(mem_hbm, msgs_hbm, dst_hbm, q_hbm, stamp_hbm, zf_hbm, out_hbm,
               dstv, qv, slotm, sq, tgtm, gsrc, oposa, oposb,
               ra0, ra1, rb0, rb1, zbuf, acc,
               s_m0, s_m1, s_g0, s_g1, s_add, s_out, s_acc, s_ld):
    c = lax.axis_index("c")
    s = lax.axis_index("s")
    cb = s * CHUNK
    h_d = pltpu.async_copy(dst_hbm.at[pl.ds(cb, CHUNK)], dstv, s_m0)
    h_q = pltpu.async_copy(q_hbm.at[pl.ds(cb, CHUNK)], qv, s_m1)
    pltpu.sync_copy(zf_hbm, zbuf)
    r0 = s * ACC_PT
    hz = [pltpu.async_copy(zbuf, acc.at[pl.ds(r0 + i * ZROWS, ZROWS)], s_acc)
          for i in range(4)]
    h_d.wait()
    h_q.wait()

    hs = []
    for k in range(KCH):
        hs.append(pltpu.async_copy(stamp_hbm.at[dstv.at[pl.ds(k * CH, CH)]],
                                   slotm.at[pl.ds(k * CH, CH)], s_ld))
        hs.append(pltpu.async_copy(stamp_hbm.at[qv.at[pl.ds(k * CH, CH)]],
                                   sq.at[pl.ds(k * CH, CH)], s_ld))
    for h in hs:
        h.wait()

    ii = lax.iota(jnp.int32, L)
    wid = s * NC + c
    dmp = DUMP + s
    odmp = B + wid

    def tcomp(v, carry):
        r, col = v // (CH // L), (v % (CH // L)) * L
        st = slotm[pl.ds(v * L, L)] - 1
        mm = 1 - ((st ^ c) & 1)
        tgtm[r, pl.ds(col, L)] = (st >> 1) * mm + dmp * (1 - mm)
        sv = sq[pl.ds(v * L, L)]
        q = qv[pl.ds(v * L, L)]
        stq = sv - 1
        pres = jnp.minimum(sv, 1)
        pos = cb + v * L + ii
        pm = pres * (1 - ((stq ^ c) & 1))
        gsrc[r, pl.ds(col, L)] = (stq >> 1) * pm + dmp * (1 - pm)
        oposa[r, pl.ds(col, L)] = pos * pm + odmp * (1 - pm)
        am = (1 - pres) * (1 - ((q ^ c) & 1))
        oposb[r, pl.ds(col, L)] = pos * am + odmp * (1 - am)
        return carry

    lax.fori_loop(0, CHUNK // L, tcomp, jnp.int32(0), unroll=8)
    for h in hz:
        h.wait()
    plsc.subcore_barrier()

    ra = [ra0, ra1]
    rb = [rb0, rb1]
    sm = [s_m0, s_m1]
    sg = [s_g0, s_g1]
    ha, hb, hadd, hout = {}, {}, {}, {}
    ha[0] = pltpu.async_copy(msgs_hbm.at[pl.ds(cb, CH)], ra0, sm[0])
    hb[0] = pltpu.async_copy(mem_hbm.at[qv.at[pl.ds(0, CH)]], rb0, sg[0])
    for k in range(KCH):
        if k >= 1:
            hadd[k - 1].wait()
            hout[k - 1].wait()
        if k + 1 < KCH:
            ha[k + 1] = pltpu.async_copy(
                msgs_hbm.at[pl.ds(cb + (k + 1) * CH, CH)],
                ra[(k + 1) % 2], sm[(k + 1) % 2])
            hb[k + 1] = pltpu.async_copy(
                mem_hbm.at[qv.at[pl.ds((k + 1) * CH, CH)]],
                rb[(k + 1) % 2], sg[(k + 1) % 2])
        ha[k].wait()
        hadd[k] = pltpu.async_copy(ra[k % 2], acc.at[tgtm.at[k]], s_add,
                                   add=True)
        hb[k].wait()
        hout[k] = pltpu.async_copy(rb[k % 2], out_hbm.at[oposb.at[k]], s_out)
    hadd[KCH - 1].wait()
    hout[KCH - 1].wait()
    plsc.subcore_barrier()

    hg, hpo = {}, {}
    hg[0] = pltpu.async_copy(acc.at[gsrc.at[0]], ra0, sm[0])
    for k in range(KCH):
        if k >= 1:
            hpo[k - 1].wait()
        if k + 1 < KCH:
            hg[k + 1] = pltpu.async_copy(acc.at[gsrc.at[k + 1]],
                                         ra[(k + 1) % 2], sm[(k + 1) % 2])
        hg[k].wait()
        hpo[k] = pltpu.async_copy(ra[k % 2], out_hbm.at[oposa.at[k]], s_out)
    hpo[KCH - 1].wait()


_main_call = functools.partial(
    pl.kernel,
    out_type=jax.ShapeDtypeStruct((B + CH, D), jnp.float32),
    mesh=_mesh,
    scratch_types=[
        pltpu.VMEM((CHUNK,), jnp.int32),
        pltpu.VMEM((CHUNK,), jnp.int32),
        pltpu.VMEM((CHUNK,), jnp.int32),
        pltpu.VMEM((CHUNK,), jnp.int32),
        pltpu.VMEM((KCH, CH), jnp.int32),
        pltpu.VMEM((KCH, CH), jnp.int32),
        pltpu.VMEM((KCH, CH), jnp.int32),
        pltpu.VMEM((KCH, CH), jnp.int32),
        pltpu.VMEM((CH, D), jnp.float32),
        pltpu.VMEM((CH, D), jnp.float32),
        pltpu.VMEM((CH, D), jnp.float32),
        pltpu.VMEM((CH, D), jnp.float32),
        pltpu.VMEM((ZROWS, D), jnp.float32),
        pltpu.VMEM_SHARED((ACC_ROWS, D), jnp.float32),
        pltpu.SemaphoreType.DMA,
        pltpu.SemaphoreType.DMA,
        pltpu.SemaphoreType.DMA,
        pltpu.SemaphoreType.DMA,
        pltpu.SemaphoreType.DMA,
        pltpu.SemaphoreType.DMA,
        pltpu.SemaphoreType.DMA,
        pltpu.SemaphoreType.DMA,
    ],
)(_main_body)


def kernel(mem, msgs, dst_ids, query_ids):
    dst_ids = dst_ids.astype(jnp.int32)
    query_ids = query_ids.astype(jnp.int32)
    zf = jnp.zeros((ZROWS, D), jnp.float32)
    stamp = _stamp_call(dst_ids)
    out = _main_call(mem, msgs, dst_ids, query_ids, stamp, zf)
    return out[:B]


# speedup vs baseline: 634.4711x; 1.2114x over previous
import functools

import jax
import jax.numpy as jnp
from jax import lax
from jax.experimental import pallas as pl
from jax.experimental.pallas import tpu as pltpu
from jax.experimental.pallas import tpu_sc as plsc

M = 100000
D = 128
N = 16384
B = 16384

NC = 2
NS = 16
L = 16
NW = NC * NS

SPT = 3136
STAMP_N = SPT * NW
CH = 64
CHUNK = N // NS
KCH = CHUNK // CH
SLOTS = N // NC
DUMP = SLOTS
ACC_PT = 520
ACC_ROWS = ACC_PT * NS
ZROWS = 8

_mesh = plsc.VectorSubcoreMesh(core_axis_name="c", subcore_axis_name="s")


def _stamp_body(dst_hbm, stamp_hbm, dstv, stampv):
    c = lax.axis_index("c")
    s = lax.axis_index("s")
    wid = s * NC + c
    base = wid * SPT
    zero = jnp.zeros((L,), jnp.int32)

    def z(i, carry):
        stampv[pl.ds(i * L, L)] = zero
        return carry

    lax.fori_loop(0, SPT // L, z, jnp.int32(0), unroll=8)
    pltpu.sync_copy(dst_hbm, dstv)
    ii = lax.iota(jnp.int32, L)

    def comp(v, carry):
        ids = dstv[pl.ds(v * L, L)]
        d = ids - base
        m = (d >= 0) & (d < SPT)
        dcl = jnp.minimum(jnp.maximum(d, 0), SPT - 1)
        plsc.store_scatter(stampv, [dcl], v * L + ii + 1, mask=m)
        return carry

    lax.fori_loop(0, N // L, comp, jnp.int32(0), unroll=8)
    pltpu.sync_copy(stampv, stamp_hbm.at[pl.ds(base, SPT)])


_stamp_call = functools.partial(
    pl.kernel,
    out_type=jax.ShapeDtypeStruct((STAMP_N,), jnp.int32),
    mesh=_mesh,
    compiler_params=pltpu.CompilerParams(needs_layout_passes=False),
    scratch_types=[
        pltpu.VMEM((N,), jnp.int32),
        pltpu.VMEM((SPT,), jnp.int32),
    ],
)(_stamp_body)


G = 64
NGR = CHUNK // G


def _main_body(mem_hbm, msgs_hbm, dst_hbm, q_hbm, stamp_hbm, zf_hbm, out_hbm,
               dstv, qv, slotm, sq, tgtm, srcP, posP, srcA, posA,
               posP2, posA2, ra0, ra1, zbuf, acc,
               s_m0, s_m1, s_g0, s_g1, s_o0, s_o1, s_add, s_acc, s_ld):
    c = lax.axis_index("c")
    s = lax.axis_index("s")
    cb = s * CHUNK
    h_d = pltpu.async_copy(dst_hbm.at[pl.ds(cb, CHUNK)], dstv, s_m0)
    h_q = pltpu.async_copy(q_hbm.at[pl.ds(cb, CHUNK)], qv, s_m1)
    pltpu.sync_copy(zf_hbm, zbuf)
    r0 = s * ACC_PT
    hz = [pltpu.async_copy(zbuf, acc.at[pl.ds(r0 + i * ZROWS, ZROWS)], s_acc)
          for i in range(ACC_PT // ZROWS)]
    h_d.wait()
    h_q.wait()

    hs = []
    for k in range(KCH):
        hs.append(pltpu.async_copy(stamp_hbm.at[dstv.at[pl.ds(k * CH, CH)]],
                                   slotm.at[pl.ds(k * CH, CH)], s_ld))
        hs.append(pltpu.async_copy(stamp_hbm.at[qv.at[pl.ds(k * CH, CH)]],
                                   sq.at[pl.ds(k * CH, CH)], s_ld))
    for h in hs:
        h.wait()

    ii = lax.iota(jnp.int32, L)
    wid = s * NC + c
    dmp = DUMP + s
    odmp = B + wid
    zv = jnp.zeros((L,), jnp.int32)
    CPR = CH // L

    def pf(i, carry):
        srcP[pl.ds(i * L, L)] = zv + dmp
        posP[pl.ds(i * L, L)] = zv + odmp
        srcA[pl.ds(i * L, L)] = zv
        posA[pl.ds(i * L, L)] = zv + odmp
        return carry

    lax.fori_loop(0, CHUNK // L, pf, jnp.int32(0), unroll=8)

    def tcomp(v, carry):
        cntP, cntA = carry
        r, col = v // CPR, (v % CPR) * L
        st = slotm[pl.ds(v * L, L)] - 1
        mm = 1 - ((st ^ c) & 1)
        tgtm[r, pl.ds(col, L)] = (st >> 1) * mm + dmp * (1 - mm)
        sv = sq[pl.ds(v * L, L)]
        q = qv[pl.ds(v * L, L)]
        stq = sv - 1
        pos = cb + v * L + ii
        presb = sv >= 1
        mineP = presb & (((stq ^ c) & 1) == 0)
        mineA = (~presb) & (((q ^ c) & 1) == 0)
        plsc.store_compressed(srcP.at[pl.ds(cntP, L)], stq >> 1, mask=mineP)
        plsc.store_compressed(posP.at[pl.ds(cntP, L)], pos, mask=mineP)
        plsc.store_compressed(srcA.at[pl.ds(cntA, L)], q, mask=mineA)
        plsc.store_compressed(posA.at[pl.ds(cntA, L)], pos, mask=mineA)
        cntP = cntP + jnp.sum(mineP.astype(jnp.int32))
        cntA = cntA + jnp.sum(mineA.astype(jnp.int32))
        return cntP, cntA

    cntP, cntA = lax.fori_loop(0, CHUNK // L, tcomp,
                               (jnp.int32(0), jnp.int32(0)), unroll=4)

    def mv(i, carry):
        posP2[i, pl.ds(0, L)] = posP[pl.ds(i * L, L)]
        posA2[i, pl.ds(0, L)] = posA[pl.ds(i * L, L)]
        return carry

    lax.fori_loop(0, CHUNK // L, mv, jnp.int32(0), unroll=8)
    for h in hz:
        h.wait()
    plsc.subcore_barrier()

    ra = [ra0, ra1]
    sm = [s_m0, s_m1]
    ha, hadd = {}, {}
    ha[0] = pltpu.async_copy(msgs_hbm.at[pl.ds(cb, CH)], ra0, sm[0])
    for k in range(KCH):
        if k >= 1:
            hadd[k - 1].wait()
        if k + 1 < KCH:
            ha[k + 1] = pltpu.async_copy(
                msgs_hbm.at[pl.ds(cb + (k + 1) * CH, CH)],
                ra[(k + 1) % 2], sm[(k + 1) % 2])
        ha[k].wait()
        hadd[k] = pltpu.async_copy(ra[k % 2], acc.at[tgtm.at[k]], s_add,
                                   add=True)
    hadd[KCH - 1].wait()

    gb = [ra0, ra1]
    sg = [s_g0, s_g1]
    so = [s_o0, s_o1]

    def stream(src_ref, srcbuf, posbuf2, n):
        ng = (n + (G - 1)) // G
        hg, hsc = {}, {}

        def fire(g):
            return pltpu.async_copy(src_ref.at[srcbuf.at[pl.ds(g * G, G)]],
                                    gb[g % 2], sg[g % 2])

        @pl.when(0 < ng)
        def _():
            hg[0] = fire(0)

        @pl.when(1 < ng)
        def _():
            hg[1] = fire(1)

        for g in range(NGR):
            @pl.when(g < ng)
            def _(g=g):
                hg[g].wait()
                hsc[g] = []
                for j in range(G // L):
                    hsc[g].append(
                        pltpu.async_copy(
                            gb[g % 2].at[pl.ds(j * L, L)],
                            out_hbm.at[posbuf2.at[g * (G // L) + j]],
                            so[g % 2]))
            if g + 2 < NGR:
                @pl.when(g + 2 < ng)
                def _(g=g):
                    for h in hsc[g]:
                        h.wait()
                    hg[g + 2] = fire(g + 2)
        for g in range(NGR):
            @pl.when((g < ng) & (g + 2 >= ng))
            def _(g=g):
                for h in hsc[g]:
                    h.wait()

    stream(mem_hbm, srcA, posA2, cntA)
    plsc.subcore_barrier()
    stream(acc, srcP, posP2, cntP)


_main_call = functools.partial(
    pl.kernel,
    out_type=jax.ShapeDtypeStruct((B + CH, D), jnp.float32),
    mesh=_mesh,
    compiler_params=pltpu.CompilerParams(needs_layout_passes=False),
    scratch_types=[
        pltpu.VMEM((CHUNK,), jnp.int32),
        pltpu.VMEM((CHUNK,), jnp.int32),
        pltpu.VMEM((CHUNK,), jnp.int32),
        pltpu.VMEM((CHUNK,), jnp.int32),
        pltpu.VMEM((KCH, CH), jnp.int32),
        pltpu.VMEM((CHUNK,), jnp.int32),
        pltpu.VMEM((CHUNK,), jnp.int32),
        pltpu.VMEM((CHUNK,), jnp.int32),
        pltpu.VMEM((CHUNK,), jnp.int32),
        pltpu.VMEM((CHUNK // L, L), jnp.int32),
        pltpu.VMEM((CHUNK // L, L), jnp.int32),
        pltpu.VMEM((CH, D), jnp.float32),
        pltpu.VMEM((CH, D), jnp.float32),
        pltpu.VMEM((ZROWS, D), jnp.float32),
        pltpu.VMEM_SHARED((ACC_ROWS, D), jnp.float32),
        pltpu.SemaphoreType.DMA,
        pltpu.SemaphoreType.DMA,
        pltpu.SemaphoreType.DMA,
        pltpu.SemaphoreType.DMA,
        pltpu.SemaphoreType.DMA,
        pltpu.SemaphoreType.DMA,
        pltpu.SemaphoreType.DMA,
        pltpu.SemaphoreType.DMA,
        pltpu.SemaphoreType.DMA,
    ],
)(_main_body)


def kernel(mem, msgs, dst_ids, query_ids):
    dst_ids = dst_ids.astype(jnp.int32)
    query_ids = query_ids.astype(jnp.int32)
    zf = jnp.zeros((ZROWS, D), jnp.float32)
    stamp = _stamp_call(dst_ids)
    out = _main_call(mem, msgs, dst_ids, query_ids, stamp, zf)
    return out[:B]


# speedup vs baseline: 671.6141x; 1.0585x over previous
import functools

import jax
import jax.numpy as jnp
from jax import lax
from jax.experimental import pallas as pl
from jax.experimental.pallas import tpu as pltpu
from jax.experimental.pallas import tpu_sc as plsc

M = 100000
D = 128
N = 16384
B = 16384

NC = 2
NS = 16
L = 16
NW = NC * NS

SPT = 3136
STAMP_N = SPT * NW
CH = 64
CHUNK = N // NS
KCH = CHUNK // CH
SLOTS = N // NC
DUMP = SLOTS
ACC_PT = 520
ACC_ROWS = ACC_PT * NS
ZROWS = 8

_mesh = plsc.VectorSubcoreMesh(core_axis_name="c", subcore_axis_name="s")


def _stamp_body(dst_hbm, stamp_hbm, dstv, stampv):
    c = lax.axis_index("c")
    s = lax.axis_index("s")
    wid = s * NC + c
    base = wid * SPT
    zero = jnp.zeros((L,), jnp.int32)

    def z(i, carry):
        stampv[pl.ds(i * L, L)] = zero
        return carry

    lax.fori_loop(0, SPT // L, z, jnp.int32(0), unroll=8)
    pltpu.sync_copy(dst_hbm, dstv)
    ii = lax.iota(jnp.int32, L)

    def comp(v, carry):
        ids = dstv[pl.ds(v * L, L)]
        d = ids - base
        m = (d >= 0) & (d < SPT)
        dcl = jnp.minimum(jnp.maximum(d, 0), SPT - 1)
        plsc.store_scatter(stampv, [dcl], v * L + ii + 1, mask=m)
        return carry

    lax.fori_loop(0, N // L, comp, jnp.int32(0), unroll=8)
    pltpu.sync_copy(stampv, stamp_hbm.at[pl.ds(base, SPT)])


_stamp_call = functools.partial(
    pl.kernel,
    out_type=jax.ShapeDtypeStruct((STAMP_N,), jnp.int32),
    mesh=_mesh,
    compiler_params=pltpu.CompilerParams(needs_layout_passes=False),
    scratch_types=[
        pltpu.VMEM((N,), jnp.int32),
        pltpu.VMEM((SPT,), jnp.int32),
    ],
)(_stamp_body)


G = 64
NGR = CHUNK // G


def _main_body(mem_hbm, msgs_hbm, dst_hbm, q_hbm, stamp_hbm, zf_hbm, out_hbm,
               dstv, qv, slotm, sq, tgtm, srcP, posP, srcA, posA,
               posP2, posA2, ra0, ra1, zbuf, acc,
               s_m0, s_m1, s_g0, s_g1, s_o0, s_o1, s_add, s_acc, s_ld):
    c = lax.axis_index("c")
    s = lax.axis_index("s")
    cb = s * CHUNK
    h_d = pltpu.async_copy(dst_hbm.at[pl.ds(cb, CHUNK)], dstv, s_m0)
    h_q = pltpu.async_copy(q_hbm.at[pl.ds(cb, CHUNK)], qv, s_m1)
    pltpu.sync_copy(zf_hbm, zbuf)
    r0 = s * ACC_PT
    hz = [pltpu.async_copy(zbuf, acc.at[pl.ds(r0 + i * ZROWS, ZROWS)], s_acc)
          for i in range(ACC_PT // ZROWS)]
    h_d.wait()
    h_q.wait()

    hs = []
    for k in range(KCH):
        hs.append(pltpu.async_copy(stamp_hbm.at[dstv.at[pl.ds(k * CH, CH)]],
                                   slotm.at[pl.ds(k * CH, CH)], s_ld))
        hs.append(pltpu.async_copy(stamp_hbm.at[qv.at[pl.ds(k * CH, CH)]],
                                   sq.at[pl.ds(k * CH, CH)], s_ld))
    for h in hs:
        h.wait()

    ii = lax.iota(jnp.int32, L)
    wid = s * NC + c
    dmp = DUMP + s
    odmp = B + wid
    zv = jnp.zeros((L,), jnp.int32)
    CPR = CH // L

    def pf(i, carry):
        srcP[pl.ds(i * L, L)] = zv + dmp
        posP[pl.ds(i * L, L)] = zv + odmp
        srcA[pl.ds(i * L, L)] = zv
        posA[pl.ds(i * L, L)] = zv + odmp
        return carry

    lax.fori_loop(0, CHUNK // L, pf, jnp.int32(0), unroll=8)

    def tcomp(v, carry):
        cntP, cntA = carry
        r, col = v // CPR, (v % CPR) * L
        st = slotm[pl.ds(v * L, L)] - 1
        mm = 1 - ((st ^ c) & 1)
        tgtm[r, pl.ds(col, L)] = (st >> 1) * mm + dmp * (1 - mm)
        sv = sq[pl.ds(v * L, L)]
        q = qv[pl.ds(v * L, L)]
        stq = sv - 1
        pos = cb + v * L + ii
        presb = sv >= 1
        mineP = presb & (((stq ^ c) & 1) == 0)
        mineA = (~presb) & (((q ^ c) & 1) == 0)
        plsc.store_compressed(srcP.at[pl.ds(cntP, L)], stq >> 1, mask=mineP)
        plsc.store_compressed(posP.at[pl.ds(cntP, L)], pos, mask=mineP)
        plsc.store_compressed(srcA.at[pl.ds(cntA, L)], q, mask=mineA)
        plsc.store_compressed(posA.at[pl.ds(cntA, L)], pos, mask=mineA)
        cntP = cntP + jnp.sum(mineP.astype(jnp.int32))
        cntA = cntA + jnp.sum(mineA.astype(jnp.int32))
        return cntP, cntA

    cntP, cntA = lax.fori_loop(0, CHUNK // L, tcomp,
                               (jnp.int32(0), jnp.int32(0)), unroll=4)

    def mv(g, carry):
        for j in range(G // L):
            posP2[g, pl.ds(j * L, L)] = posP[pl.ds(g * G + j * L, L)]
            posA2[g, pl.ds(j * L, L)] = posA[pl.ds(g * G + j * L, L)]
        return carry

    lax.fori_loop(0, NGR, mv, jnp.int32(0), unroll=4)
    for h in hz:
        h.wait()
    plsc.subcore_barrier()

    ra = [ra0, ra1]
    sm = [s_m0, s_m1]
    ha, hadd = {}, {}
    ha[0] = pltpu.async_copy(msgs_hbm.at[pl.ds(cb, CH)], ra0, sm[0])
    for k in range(KCH):
        if k >= 1:
            hadd[k - 1].wait()
        if k + 1 < KCH:
            ha[k + 1] = pltpu.async_copy(
                msgs_hbm.at[pl.ds(cb + (k + 1) * CH, CH)],
                ra[(k + 1) % 2], sm[(k + 1) % 2])
        ha[k].wait()
        hadd[k] = pltpu.async_copy(ra[k % 2], acc.at[tgtm.at[k]], s_add,
                                   add=True)
    hadd[KCH - 1].wait()

    gb = [ra0, ra1]
    sg = [s_g0, s_g1]
    so = [s_o0, s_o1]

    def stream(src_ref, srcbuf, posbuf2, n):
        ng = (n + (G - 1)) // G
        hg, hsc = {}, {}

        def fire(g):
            return pltpu.async_copy(src_ref.at[srcbuf.at[pl.ds(g * G, G)]],
                                    gb[g % 2], sg[g % 2])

        @pl.when(0 < ng)
        def _():
            hg[0] = fire(0)

        @pl.when(1 < ng)
        def _():
            hg[1] = fire(1)

        for g in range(NGR):
            @pl.when(g < ng)
            def _(g=g):
                hg[g].wait()
                hsc[g] = [pltpu.async_copy(gb[g % 2],
                                           out_hbm.at[posbuf2.at[g]],
                                           so[g % 2])]
            if g + 2 < NGR:
                @pl.when(g + 2 < ng)
                def _(g=g):
                    for h in hsc[g]:
                        h.wait()
                    hg[g + 2] = fire(g + 2)
        for g in range(NGR):
            @pl.when((g < ng) & (g + 2 >= ng))
            def _(g=g):
                for h in hsc[g]:
                    h.wait()

    stream(mem_hbm, srcA, posA2, cntA)
    plsc.subcore_barrier()
    stream(acc, srcP, posP2, cntP)


_main_call = functools.partial(
    pl.kernel,
    out_type=jax.ShapeDtypeStruct((B + CH, D), jnp.float32),
    mesh=_mesh,
    compiler_params=pltpu.CompilerParams(needs_layout_passes=False),
    scratch_types=[
        pltpu.VMEM((CHUNK,), jnp.int32),
        pltpu.VMEM((CHUNK,), jnp.int32),
        pltpu.VMEM((CHUNK,), jnp.int32),
        pltpu.VMEM((CHUNK,), jnp.int32),
        pltpu.VMEM((KCH, CH), jnp.int32),
        pltpu.VMEM((CHUNK,), jnp.int32),
        pltpu.VMEM((CHUNK,), jnp.int32),
        pltpu.VMEM((CHUNK,), jnp.int32),
        pltpu.VMEM((CHUNK,), jnp.int32),
        pltpu.VMEM((NGR, G), jnp.int32),
        pltpu.VMEM((NGR, G), jnp.int32),
        pltpu.VMEM((CH, D), jnp.float32),
        pltpu.VMEM((CH, D), jnp.float32),
        pltpu.VMEM((ZROWS, D), jnp.float32),
        pltpu.VMEM_SHARED((ACC_ROWS, D), jnp.float32),
        pltpu.SemaphoreType.DMA,
        pltpu.SemaphoreType.DMA,
        pltpu.SemaphoreType.DMA,
        pltpu.SemaphoreType.DMA,
        pltpu.SemaphoreType.DMA,
        pltpu.SemaphoreType.DMA,
        pltpu.SemaphoreType.DMA,
        pltpu.SemaphoreType.DMA,
        pltpu.SemaphoreType.DMA,
    ],
)(_main_body)


def kernel(mem, msgs, dst_ids, query_ids):
    dst_ids = dst_ids.astype(jnp.int32)
    query_ids = query_ids.astype(jnp.int32)
    zf = jnp.zeros((ZROWS, D), jnp.float32)
    stamp = _stamp_call(dst_ids)
    out = _main_call(mem, msgs, dst_ids, query_ids, stamp, zf)
    return out[:B]


# speedup vs baseline: 675.0007x; 1.0050x over previous
import functools

import jax
import jax.numpy as jnp
from jax import lax
from jax.experimental import pallas as pl
from jax.experimental.pallas import tpu as pltpu
from jax.experimental.pallas import tpu_sc as plsc

M = 100000
D = 128
N = 16384
B = 16384

NC = 2
NS = 16
L = 16
NW = NC * NS

SPT = 3136
STAMP_N = SPT * NW
CH = 64
CHUNK = N // NS
KCH = CHUNK // CH
SLOTS = N // NC
DUMP = SLOTS
ACC_PT = 520
ACC_ROWS = ACC_PT * NS
ZROWS = 8

_mesh = plsc.VectorSubcoreMesh(core_axis_name="c", subcore_axis_name="s")


def _stamp_body(dst_hbm, stamp_hbm, dstv, stampv):
    c = lax.axis_index("c")
    s = lax.axis_index("s")
    wid = s * NC + c
    base = wid * SPT
    zero = jnp.zeros((L,), jnp.int32)

    def z(i, carry):
        stampv[pl.ds(i * L, L)] = zero
        return carry

    lax.fori_loop(0, SPT // L, z, jnp.int32(0), unroll=8)
    pltpu.sync_copy(dst_hbm, dstv)
    ii = lax.iota(jnp.int32, L)

    def comp(v, carry):
        ids = dstv[pl.ds(v * L, L)]
        d = ids - base
        m = (d >= 0) & (d < SPT)
        dcl = jnp.minimum(jnp.maximum(d, 0), SPT - 1)
        plsc.store_scatter(stampv, [dcl], v * L + ii + 1, mask=m)
        return carry

    lax.fori_loop(0, N // L, comp, jnp.int32(0), unroll=8)
    pltpu.sync_copy(stampv, stamp_hbm.at[pl.ds(base, SPT)])


_stamp_call = functools.partial(
    pl.kernel,
    out_type=jax.ShapeDtypeStruct((STAMP_N,), jnp.int32),
    mesh=_mesh,
    compiler_params=pltpu.CompilerParams(needs_layout_passes=False),
    scratch_types=[
        pltpu.VMEM((N,), jnp.int32),
        pltpu.VMEM((SPT,), jnp.int32),
    ],
)(_stamp_body)


G = 64
NGR = CHUNK // G


def _main_body(mem_hbm, msgs_hbm, dst_hbm, q_hbm, stamp_hbm, zf_hbm, out_hbm,
               dstv, qv, slotm, sq, tgtm, srcP, posP, srcA, posA,
               posP2, posA2, ra0, ra1, zbuf, acc,
               s_m0, s_m1, s_g0, s_g1, s_o0, s_o1, s_add, s_acc, s_ld):
    c = lax.axis_index("c")
    s = lax.axis_index("s")
    cb = s * CHUNK
    h_d = pltpu.async_copy(dst_hbm.at[pl.ds(cb, CHUNK)], dstv, s_m0)
    h_q = pltpu.async_copy(q_hbm.at[pl.ds(cb, CHUNK)], qv, s_m1)
    pltpu.sync_copy(zf_hbm, zbuf)
    r0 = s * ACC_PT
    hz = [pltpu.async_copy(zbuf, acc.at[pl.ds(r0 + i * ZROWS, ZROWS)], s_acc)
          for i in range(ACC_PT // ZROWS)]
    h_d.wait()
    h_q.wait()

    h_s1 = pltpu.async_copy(stamp_hbm.at[dstv.at[pl.ds(0, CHUNK)]],
                            slotm, s_ld)
    h_s2 = pltpu.async_copy(stamp_hbm.at[qv.at[pl.ds(0, CHUNK)]],
                            sq, s_ld)
    h_s1.wait()
    h_s2.wait()

    ii = lax.iota(jnp.int32, L)
    wid = s * NC + c
    dmp = DUMP + s
    odmp = B + wid
    zv = jnp.zeros((L,), jnp.int32)
    CPR = CH // L

    def pf(i, carry):
        srcP[pl.ds(i * L, L)] = zv + dmp
        posP[pl.ds(i * L, L)] = zv + odmp
        srcA[pl.ds(i * L, L)] = zv
        posA[pl.ds(i * L, L)] = zv + odmp
        return carry

    lax.fori_loop(0, CHUNK // L, pf, jnp.int32(0), unroll=8)

    def tcomp(v, carry):
        cntP, cntA = carry
        r, col = v // CPR, (v % CPR) * L
        st = slotm[pl.ds(v * L, L)] - 1
        mm = 1 - ((st ^ c) & 1)
        tgtm[r, pl.ds(col, L)] = (st >> 1) * mm + dmp * (1 - mm)
        sv = sq[pl.ds(v * L, L)]
        q = qv[pl.ds(v * L, L)]
        stq = sv - 1
        pos = cb + v * L + ii
        presb = sv >= 1
        mineP = presb & (((stq ^ c) & 1) == 0)
        mineA = (~presb) & (((q ^ c) & 1) == 0)
        plsc.store_compressed(srcP.at[pl.ds(cntP, L)], stq >> 1, mask=mineP)
        plsc.store_compressed(posP.at[pl.ds(cntP, L)], pos, mask=mineP)
        plsc.store_compressed(srcA.at[pl.ds(cntA, L)], q, mask=mineA)
        plsc.store_compressed(posA.at[pl.ds(cntA, L)], pos, mask=mineA)
        cntP = cntP + jnp.sum(mineP.astype(jnp.int32))
        cntA = cntA + jnp.sum(mineA.astype(jnp.int32))
        return cntP, cntA

    cntP, cntA = lax.fori_loop(0, CHUNK // L, tcomp,
                               (jnp.int32(0), jnp.int32(0)), unroll=4)

    def mv(g, carry):
        for j in range(G // L):
            posP2[g, pl.ds(j * L, L)] = posP[pl.ds(g * G + j * L, L)]
            posA2[g, pl.ds(j * L, L)] = posA[pl.ds(g * G + j * L, L)]
        return carry

    lax.fori_loop(0, NGR, mv, jnp.int32(0), unroll=4)
    for h in hz:
        h.wait()
    plsc.subcore_barrier()

    ra = [ra0, ra1]
    sm = [s_m0, s_m1]
    ha, hadd = {}, {}
    ha[0] = pltpu.async_copy(msgs_hbm.at[pl.ds(cb, CH)], ra0, sm[0])
    for k in range(KCH):
        if k >= 1:
            hadd[k - 1].wait()
        if k + 1 < KCH:
            ha[k + 1] = pltpu.async_copy(
                msgs_hbm.at[pl.ds(cb + (k + 1) * CH, CH)],
                ra[(k + 1) % 2], sm[(k + 1) % 2])
        ha[k].wait()
        hadd[k] = pltpu.async_copy(ra[k % 2], acc.at[tgtm.at[k]], s_add,
                                   add=True)
    hadd[KCH - 1].wait()

    gb = [ra0, ra1]
    sg = [s_g0, s_g1]
    so = [s_o0, s_o1]

    def stream(src_ref, srcbuf, posbuf2, n):
        ng = (n + (G - 1)) // G
        hg, hsc = {}, {}

        def fire(g):
            return pltpu.async_copy(src_ref.at[srcbuf.at[pl.ds(g * G, G)]],
                                    gb[g % 2], sg[g % 2])

        @pl.when(0 < ng)
        def _():
            hg[0] = fire(0)

        @pl.when(1 < ng)
        def _():
            hg[1] = fire(1)

        for g in range(NGR):
            @pl.when(g < ng)
            def _(g=g):
                hg[g].wait()
                hsc[g] = [pltpu.async_copy(gb[g % 2],
                                           out_hbm.at[posbuf2.at[g]],
                                           so[g % 2])]
            if g + 2 < NGR:
                @pl.when(g + 2 < ng)
                def _(g=g):
                    for h in hsc[g]:
                        h.wait()
                    hg[g + 2] = fire(g + 2)
        for g in range(NGR):
            @pl.when((g < ng) & (g + 2 >= ng))
            def _(g=g):
                for h in hsc[g]:
                    h.wait()

    stream(mem_hbm, srcA, posA2, cntA)
    plsc.subcore_barrier()
    stream(acc, srcP, posP2, cntP)


_main_call = functools.partial(
    pl.kernel,
    out_type=jax.ShapeDtypeStruct((B + CH, D), jnp.float32),
    mesh=_mesh,
    compiler_params=pltpu.CompilerParams(needs_layout_passes=False),
    scratch_types=[
        pltpu.VMEM((CHUNK,), jnp.int32),
        pltpu.VMEM((CHUNK,), jnp.int32),
        pltpu.VMEM((CHUNK,), jnp.int32),
        pltpu.VMEM((CHUNK,), jnp.int32),
        pltpu.VMEM((KCH, CH), jnp.int32),
        pltpu.VMEM((CHUNK,), jnp.int32),
        pltpu.VMEM((CHUNK,), jnp.int32),
        pltpu.VMEM((CHUNK,), jnp.int32),
        pltpu.VMEM((CHUNK,), jnp.int32),
        pltpu.VMEM((NGR, G), jnp.int32),
        pltpu.VMEM((NGR, G), jnp.int32),
        pltpu.VMEM((CH, D), jnp.float32),
        pltpu.VMEM((CH, D), jnp.float32),
        pltpu.VMEM((ZROWS, D), jnp.float32),
        pltpu.VMEM_SHARED((ACC_ROWS, D), jnp.float32),
        pltpu.SemaphoreType.DMA,
        pltpu.SemaphoreType.DMA,
        pltpu.SemaphoreType.DMA,
        pltpu.SemaphoreType.DMA,
        pltpu.SemaphoreType.DMA,
        pltpu.SemaphoreType.DMA,
        pltpu.SemaphoreType.DMA,
        pltpu.SemaphoreType.DMA,
        pltpu.SemaphoreType.DMA,
    ],
)(_main_body)


def kernel(mem, msgs, dst_ids, query_ids):
    dst_ids = dst_ids.astype(jnp.int32)
    query_ids = query_ids.astype(jnp.int32)
    zf = jnp.zeros((ZROWS, D), jnp.float32)
    stamp = _stamp_call(dst_ids)
    out = _main_call(mem, msgs, dst_ids, query_ids, stamp, zf)
    return out[:B]
